# baseline (speedup 1.0000x reference)
import numpy as np
import jax
import jax.numpy as jnp
from jax.experimental import pallas as pl
from jax.experimental.pallas import tpu as pltpu

_SLOW_FRAMES = 8


def _slow_indices(t):
    return tuple(int(v) for v in np.linspace(0, t - 1, _SLOW_FRAMES).astype(np.int32))


def _pack_dma_kernel(idx_tuple, frames_ref, slow_ref, fast_ref, fast_sem, slow_sem):
    fast_cp = pltpu.make_async_copy(frames_ref, fast_ref, fast_sem)
    fast_cp.start()
    slow_cps = []
    for j, t in enumerate(idx_tuple):
        cp = pltpu.make_async_copy(frames_ref.at[:, t], slow_ref.at[:, j], slow_sem)
        cp.start()
        slow_cps.append(cp)
    for cp in slow_cps:
        cp.wait()
    fast_cp.wait()


def kernel(frames):
    b, t, c, h, w = frames.shape
    idx = _slow_indices(t)
    import functools
    slow, fast = pl.pallas_call(
        functools.partial(_pack_dma_kernel, idx),
        out_shape=(
            jax.ShapeDtypeStruct((b, _SLOW_FRAMES, c, h, w), frames.dtype),
            jax.ShapeDtypeStruct((b, t, c, h, w), frames.dtype),
        ),
        in_specs=[pl.BlockSpec(memory_space=pl.ANY)],
        out_specs=(
            pl.BlockSpec(memory_space=pl.ANY),
            pl.BlockSpec(memory_space=pl.ANY),
        ),
        scratch_shapes=[pltpu.SemaphoreType.DMA, pltpu.SemaphoreType.DMA],
    )(frames)
    return (slow, fast)


# speedup vs baseline: 3.9283x; 3.9283x over previous
import functools
import numpy as np
import jax
import jax.numpy as jnp
from jax.experimental import pallas as pl
from jax.experimental.pallas import tpu as pltpu

_SLOW_FRAMES = 8


def _slow_indices(t):
    return tuple(int(v) for v in np.linspace(0, t - 1, _SLOW_FRAMES).astype(np.int32))


def _gather_rows_kernel(src_rows, frames_ref, slow_ref, sem):
    cps = []
    for dst, src in enumerate(src_rows):
        cp = pltpu.make_async_copy(frames_ref.at[src], slow_ref.at[dst], sem)
        cp.start()
        cps.append(cp)
    for cp in cps:
        cp.wait()


def kernel(frames):
    b, t, c, h, w = frames.shape
    idx = _slow_indices(t)
    row = c * h * w
    frames2 = frames.reshape(b * t, row)
    src_rows = tuple(bi * t + ti for bi in range(b) for ti in idx)
    slow2 = pl.pallas_call(
        functools.partial(_gather_rows_kernel, src_rows),
        out_shape=jax.ShapeDtypeStruct((b * _SLOW_FRAMES, row), frames.dtype),
        in_specs=[pl.BlockSpec(memory_space=pl.ANY)],
        out_specs=pl.BlockSpec(memory_space=pl.ANY),
        scratch_shapes=[pltpu.SemaphoreType.DMA],
    )(frames2)
    slow = slow2.reshape(b, _SLOW_FRAMES, c, h, w)
    return (slow, frames)


# speedup vs baseline: 17.7235x; 4.5117x over previous
import numpy as np
import jax
import jax.numpy as jnp
from jax.experimental import pallas as pl
from jax.experimental.pallas import tpu as pltpu

_SLOW_FRAMES = 8


def _slow_indices(t):
    return tuple(int(v) for v in np.linspace(0, t - 1, _SLOW_FRAMES).astype(np.int32))


def _copy_block_kernel(src_ref, in_ref, out_ref):
    del src_ref
    out_ref[...] = in_ref[...]


def kernel(frames):
    b, t, c, h, w = frames.shape
    idx = _slow_indices(t)
    n_slow = len(idx)
    row = c * h * w
    sub = row // 128
    frames3 = frames.reshape(b * t, sub, 128)
    src_rows = jnp.asarray(
        [bi * t + ti for bi in range(b) for ti in idx], dtype=jnp.int32
    )

    grid_spec = pltpu.PrefetchScalarGridSpec(
        num_scalar_prefetch=1,
        grid=(b * n_slow,),
        in_specs=[
            pl.BlockSpec((1, sub, 128), lambda i, src_ref: (src_ref[i], 0, 0)),
        ],
        out_specs=pl.BlockSpec((1, sub, 128), lambda i, src_ref: (i, 0, 0)),
    )
    slow3 = pl.pallas_call(
        _copy_block_kernel,
        grid_spec=grid_spec,
        out_shape=jax.ShapeDtypeStruct((b * n_slow, sub, 128), frames.dtype),
    )(src_rows, frames3)
    slow = slow3.reshape(b, n_slow, c, h, w)
    return (slow, frames)


# speedup vs baseline: 48.2240x; 2.7209x over previous
import numpy as np
import jax
import jax.numpy as jnp
from jax.experimental import pallas as pl
from jax.experimental.pallas import tpu as pltpu

_SLOW_FRAMES = 8


def _slow_indices(t):
    return tuple(int(v) for v in np.linspace(0, t - 1, _SLOW_FRAMES).astype(np.int32))


def _copy_block_kernel(idx_ref, in_ref, out_ref):
    del idx_ref
    out_ref[...] = in_ref[...]


def kernel(frames):
    b, t, c, h, w = frames.shape
    idx = jnp.asarray(_slow_indices(t), dtype=jnp.int32)
    n_slow = _SLOW_FRAMES

    grid_spec = pltpu.PrefetchScalarGridSpec(
        num_scalar_prefetch=1,
        grid=(n_slow,),
        in_specs=[
            pl.BlockSpec(
                (b, 1, c, h, w), lambda j, idx_ref: (0, idx_ref[j], 0, 0, 0)
            ),
        ],
        out_specs=pl.BlockSpec(
            (b, 1, c, h, w), lambda j, idx_ref: (0, j, 0, 0, 0)
        ),
    )
    slow = pl.pallas_call(
        _copy_block_kernel,
        grid_spec=grid_spec,
        out_shape=jax.ShapeDtypeStruct((b, n_slow, c, h, w), frames.dtype),
    )(idx, frames)
    return (slow, frames)
